# baseline (speedup 1.0000x reference)
import jax
import jax.numpy as jnp
from jax import lax
from jax.experimental import pallas as pl
from jax.experimental.pallas import tpu as pltpu
from jax.experimental.pallas import tpu_sc as plsc

_K = 10


_B = 4


def _rerank_body(pre_smem, pre_row_ref, scores_row_ref, ids_row_ref, x_ref,
                 tail_ref, out_ref, head_vmem):
    M = x_ref.shape[1]
    q = pl.program_id(0)
    t = pl.program_id(1)
    nq = pl.num_programs(0)

    @pl.when(t == 0)
    def _compute():
        _rerank_queries(pre_smem, pre_row_ref, scores_row_ref, ids_row_ref,
                        x_ref, head_vmem, q)

    @pl.when(q < nq - 1)
    def _tail():
        out_ref[...] = tail_ref[...]

    @pl.when(q == nq - 1)
    def _head():
        out_ref[...] = head_vmem[...]


def _rerank_queries(pre_smem, pre_row_ref, scores_row_ref, ids_row_ref,
                    x_ref, head_vmem, q):
    M = x_ref.shape[1]
    Qw = head_vmem.shape[1]

    iota_r = jax.lax.broadcasted_iota(jnp.int32, (M, M), 1)
    iota_c = jax.lax.broadcasted_iota(jnp.int32, (M, M), 0)
    eid = iota_r == iota_c
    tie = iota_r < iota_c
    lane_q = jax.lax.broadcasted_iota(jnp.int32, (M, Qw), 1)

    def t_row_to_col(row, zero):
        return jnp.sum(jnp.where(eid, row, zero), axis=1, keepdims=True)

    cur = head_vmem[...]
    for b in range(_B):
        x = x_ref[b]

        X1 = x_ref[b, pl.ds(pre_smem[b, 0, 0], 1), :]
        for k in range(1, _K):
            X1 = jnp.maximum(X1, x_ref[b, pl.ds(pre_smem[b, 0, k], 1), :])

        xr = x.astype(jnp.bfloat16).astype(jnp.float32)
        X1r = X1.astype(jnp.bfloat16).astype(jnp.float32)
        s_col = jnp.sum(xr * X1r, axis=1, keepdims=True)

        v_row = scores_row_ref[b]
        ids_row = ids_row_ref[b]
        pre_row = pre_row_ref[b]

        v_col = t_row_to_col(v_row, 0.0)

        cnt1 = (v_row > v_col) | ((v_row == v_col) & tie)
        rank1_col = jnp.sum(cnt1.astype(jnp.int32), axis=1, keepdims=True)

        sorted_v_row = jnp.sum(jnp.where(rank1_col == iota_r, v_col, 0.0),
                               axis=0, keepdims=True)

        s_g_row = jnp.sum(jnp.where(iota_c == pre_row, s_col, 0.0),
                          axis=0, keepdims=True)

        r_row = (sorted_v_row + s_g_row) * 0.5
        r_col = t_row_to_col(r_row, 0.0)

        cnt2 = (r_col > r_row) | ((r_col == r_row) & (iota_c < iota_r))
        rank2_row = jnp.sum(cnt2.astype(jnp.int32), axis=0, keepdims=True)

        out_col = jnp.sum(jnp.where(rank2_row == iota_c, ids_row, 0),
                          axis=1, keepdims=True)
        cur = jnp.where(lane_q == q * _B + b, out_col, cur)
    head_vmem[...] = cur


def _make_sc_copy(N, Q, M):
    info = plsc.get_sparse_core_info()
    nw = info.num_cores * info.num_subcores
    rows = N - M
    per_w = ((rows + nw - 1) // nw + 7) // 8 * 8
    last_w = rows - (nw - 1) * per_w
    assert last_w > 0 and last_w % 8 == 0 and M % 8 == 0
    half, last_half = per_w // 2, last_w // 2
    mesh = plsc.VectorSubcoreMesh(core_axis_name="c", subcore_axis_name="s")

    def body(ranks_hbm, out_hbm, buf):
        wid = lax.axis_index("s") * info.num_cores + lax.axis_index("c")
        base = M + wid * per_w

        @pl.when(wid < nw - 1)
        def _():
            for c in range(2):
                start = base + c * half
                pltpu.sync_copy(ranks_hbm.at[pl.ds(start, half), :], buf)
                pltpu.sync_copy(buf, out_hbm.at[pl.ds(start, half), :])

        @pl.when(wid == nw - 1)
        def _():
            for c in range(2):
                start = base + c * last_half
                pltpu.sync_copy(ranks_hbm.at[pl.ds(start, last_half), :],
                                buf.at[pl.ds(0, last_half), :])
                pltpu.sync_copy(buf.at[pl.ds(0, last_half), :],
                                out_hbm.at[pl.ds(start, last_half), :])

    return pl.kernel(
        body,
        out_type=jax.ShapeDtypeStruct((N, Q), jnp.int32),
        mesh=mesh,
        scratch_types=[pltpu.VMEM((half, Q), jnp.int32)],
    )


def _assemble_body(head_ref, full_ref, out_ref):
    out_ref[...] = head_ref[...].T


def kernel(ranks, rerank_dba_final, res_top1000_dba, ranks_trans_1000_pre,
           x_dba):
    Q, M = ranks_trans_1000_pre.shape
    N = ranks.shape[0]
    D = x_dba.shape[2]
    pre3 = ranks_trans_1000_pre.reshape(Q, 1, M)
    scores3 = res_top1000_dba.reshape(Q, 1, M)
    ids3 = rerank_dba_final.reshape(Q, 1, M)
    ng = Q // _B
    nblk = N // M
    assert (ng - 1) * _B == nblk - 1 and N % M == 0

    def tail_map(q, t):
        return (jnp.where(q == ng - 1, 0, _B * q + t + 1), 0)

    return pl.pallas_call(
        _rerank_body,
        grid=(ng, _B),
        in_specs=[
            pl.BlockSpec((_B, 1, M), lambda q, t: (q, 0, 0),
                         memory_space=pltpu.SMEM),
            pl.BlockSpec((_B, 1, M), lambda q, t: (q, 0, 0)),
            pl.BlockSpec((_B, 1, M), lambda q, t: (q, 0, 0)),
            pl.BlockSpec((_B, 1, M), lambda q, t: (q, 0, 0)),
            pl.BlockSpec((_B, M, D), lambda q, t: (q, 0, 0)),
            pl.BlockSpec((M, Q), tail_map),
        ],
        out_specs=pl.BlockSpec((M, Q), tail_map),
        out_shape=jax.ShapeDtypeStruct((N, Q), jnp.int32),
        scratch_shapes=[pltpu.VMEM((M, Q), jnp.int32)],
    )(pre3, pre3, scores3, ids3, x_dba, ranks)


# speedup vs baseline: 1.7812x; 1.7812x over previous
import jax
import jax.numpy as jnp
from jax import lax
from jax.experimental import pallas as pl
from jax.experimental.pallas import tpu as pltpu
from jax.experimental.pallas import tpu_sc as plsc

_K = 10


_B = 4


def _rerank_body(pre_smem, pre_row_ref, scores_row_ref, ids_row_ref, x_ref,
                 out_ref):
    M = x_ref.shape[1]

    iota_r = jax.lax.broadcasted_iota(jnp.int32, (M, M), 1)
    iota_c = jax.lax.broadcasted_iota(jnp.int32, (M, M), 0)
    eid = iota_r == iota_c
    tie = iota_r < iota_c

    def t_row_to_col(row, zero):
        return jnp.sum(jnp.where(eid, row, zero), axis=1, keepdims=True)

    for b in range(_B):
        x = x_ref[b]

        X1 = x_ref[b, pl.ds(pre_smem[b, 0, 0], 1), :]
        for k in range(1, _K):
            X1 = jnp.maximum(X1, x_ref[b, pl.ds(pre_smem[b, 0, k], 1), :])

        xr = x.astype(jnp.bfloat16).astype(jnp.float32)
        X1r = X1.astype(jnp.bfloat16).astype(jnp.float32)
        s_col = jnp.sum(xr * X1r, axis=1, keepdims=True)

        v_row = scores_row_ref[b]
        ids_row = ids_row_ref[b]
        pre_row = pre_row_ref[b]

        v_col = t_row_to_col(v_row, 0.0)

        cnt1 = (v_row > v_col) | ((v_row == v_col) & tie)
        rank1_col = jnp.sum(cnt1.astype(jnp.int32), axis=1, keepdims=True)

        sorted_v_row = jnp.sum(jnp.where(rank1_col == iota_r, v_col, 0.0),
                               axis=0, keepdims=True)

        s_g_row = jnp.sum(jnp.where(iota_c == pre_row, s_col, 0.0),
                          axis=0, keepdims=True)

        r_row = (sorted_v_row + s_g_row) * 0.5
        r_col = t_row_to_col(r_row, 0.0)

        cnt2 = (r_row > r_col) | ((r_row == r_col) & tie)
        rank2_col = jnp.sum(cnt2.astype(jnp.int32), axis=1, keepdims=True)

        ids_col = t_row_to_col(ids_row, 0)
        out_row = jnp.sum(jnp.where(rank2_col == iota_r, ids_col, 0),
                          axis=0, keepdims=True)
        out_ref[b] = out_row


def _make_sc_copy(N, Q, M):
    info = plsc.get_sparse_core_info()
    nw = info.num_cores * info.num_subcores
    rows = N - M
    per_w = ((rows + nw - 1) // nw + 7) // 8 * 8
    last_w = rows - (nw - 1) * per_w
    assert last_w > 0 and last_w % 8 == 0 and M % 8 == 0
    half, last_half = per_w // 2, last_w // 2
    mesh = plsc.VectorSubcoreMesh(core_axis_name="c", subcore_axis_name="s")

    def body(ranks_hbm, out_hbm, buf):
        wid = lax.axis_index("s") * info.num_cores + lax.axis_index("c")
        base = M + wid * per_w

        @pl.when(wid < nw - 1)
        def _():
            for c in range(2):
                start = base + c * half
                pltpu.sync_copy(ranks_hbm.at[pl.ds(start, half), :], buf)
                pltpu.sync_copy(buf, out_hbm.at[pl.ds(start, half), :])

        @pl.when(wid == nw - 1)
        def _():
            for c in range(2):
                start = base + c * last_half
                pltpu.sync_copy(ranks_hbm.at[pl.ds(start, last_half), :],
                                buf.at[pl.ds(0, last_half), :])
                pltpu.sync_copy(buf.at[pl.ds(0, last_half), :],
                                out_hbm.at[pl.ds(start, last_half), :])

    return pl.kernel(
        body,
        out_type=jax.ShapeDtypeStruct((N, Q), jnp.int32),
        mesh=mesh,
        scratch_types=[pltpu.VMEM((half, Q), jnp.int32)],
    )


def _assemble_body(head_ref, full_ref, out_ref):
    out_ref[...] = head_ref[...].T


def kernel(ranks, rerank_dba_final, res_top1000_dba, ranks_trans_1000_pre,
           x_dba):
    Q, M = ranks_trans_1000_pre.shape
    N = ranks.shape[0]
    D = x_dba.shape[2]
    pre3 = ranks_trans_1000_pre.reshape(Q, 1, M)
    scores3 = res_top1000_dba.reshape(Q, 1, M)
    ids3 = rerank_dba_final.reshape(Q, 1, M)
    tail = _make_sc_copy(N, Q, M)(ranks)
    out3 = pl.pallas_call(
        _rerank_body,
        grid=(Q // _B,),
        in_specs=[
            pl.BlockSpec((_B, 1, M), lambda q: (q, 0, 0),
                         memory_space=pltpu.SMEM),
            pl.BlockSpec((_B, 1, M), lambda q: (q, 0, 0)),
            pl.BlockSpec((_B, 1, M), lambda q: (q, 0, 0)),
            pl.BlockSpec((_B, 1, M), lambda q: (q, 0, 0)),
            pl.BlockSpec((_B, M, D), lambda q: (q, 0, 0)),
        ],
        out_specs=pl.BlockSpec((_B, 1, M), lambda q: (q, 0, 0)),
        out_shape=jax.ShapeDtypeStruct((Q, 1, M), jnp.int32),
    )(pre3, pre3, scores3, ids3, x_dba)

    return pl.pallas_call(
        _assemble_body,
        grid=(1,),
        in_specs=[
            pl.BlockSpec((Q, M), lambda i: (0, 0)),
            pl.BlockSpec(memory_space=pl.ANY),
        ],
        out_specs=pl.BlockSpec((M, Q), lambda i: (0, 0)),
        out_shape=jax.ShapeDtypeStruct((N, Q), jnp.int32),
        input_output_aliases={1: 0},
    )(out3.reshape(Q, M), tail)
